# baseline (speedup 1.0000x reference)
import dataclasses
import functools

import jax
import jax.numpy as jnp
from jax import lax
from jax.experimental import pallas as pl
from jax.experimental.pallas import tpu as pltpu
from jax.experimental.pallas import tpu_sc as plsc

N = 10000
E = 320000
H1 = 64
H2 = 100
NUM_CLASSES = 40
ALPHA = 0.5

NCORE = 2
NSUB = 16
NTAB = 10240
RPS = NTAB // NSUB
KPT = 4
CE = 2048
CEA = 8192
UN = 16
EPS = 20480
EPAD = EPS * NSUB

RB = 1000
GRID = N // RB


def _sc_params():
    cp = pltpu.CompilerParams()
    if "needs_layout_passes" in pltpu.CompilerParams.__dataclass_fields__:
        cp = dataclasses.replace(cp, needs_layout_passes=False)
    return cp


def _deg_body(sidx, zvec, out, hist, red, tmp, hsh, ebuf, isem):
    c = lax.axis_index("c")
    s = lax.axis_index("s")
    ones16 = jnp.ones((16,), jnp.float32)
    pltpu.sync_copy(zvec, hist)
    base = s * EPS

    def chunk(j, carry):
        pltpu.async_copy(sidx.at[c, pl.ds(base + j * CE, CE)], ebuf, isem).wait()

        def vreg(i, carry2):
            iv = ebuf[pl.ds(i * 16, 16)]
            plsc.addupdate_scatter(hist, [iv], ones16)
            return carry2

        lax.fori_loop(0, CE // 16, vreg, 0)
        return carry

    lax.fori_loop(0, EPS // CE, chunk, 0)

    pltpu.sync_copy(hist, hsh.at[s])
    plsc.subcore_barrier()

    pltpu.sync_copy(zvec.at[pl.ds(s * RPS, RPS)], red)
    for t in range(NSUB):
        pltpu.sync_copy(hsh.at[t, pl.ds(s * RPS, RPS)], tmp)

        def addv(i, carry):
            red[pl.ds(i * 16, 16)] = red[pl.ds(i * 16, 16)] + tmp[pl.ds(i * 16, 16)]
            return carry

        lax.fori_loop(0, RPS // 16, addv, 0)
    pltpu.sync_copy(red, out.at[c, pl.ds(s * RPS, RPS)])


def _agg_body(gt, gidx, sidx, zvec, out, g0, g1, g2, g3, a0, a1, a2, a3,
              gbuf, sbuf, isem):
    gk = (g0, g1, g2, g3)
    ak = (a0, a1, a2, a3)
    c = lax.axis_index("c")
    s = lax.axis_index("s")
    for k in range(KPT):
        pltpu.sync_copy(gt.at[c, s, k], gk[k])
        pltpu.sync_copy(zvec, ak[k])

    def chunk(j, carry):
        cp1 = pltpu.async_copy(gidx.at[c, pl.ds(j * CEA, CEA)], gbuf, isem)
        cp2 = pltpu.async_copy(sidx.at[c, pl.ds(j * CEA, CEA)], sbuf, isem)
        cp1.wait()
        cp2.wait()

        @plsc.parallel_loop(0, CEA // 16, unroll=UN)
        def vreg(i):
            ivg = gbuf[pl.ds(i * 16, 16)]
            ivs = sbuf[pl.ds(i * 16, 16)]
            for k in range(KPT):
                v = plsc.load_gather(gk[k], [ivg])
                plsc.addupdate_scatter(ak[k], [ivs], v)

        return carry

    lax.fori_loop(0, EPAD // CEA, chunk, 0)

    for k in range(KPT):
        pltpu.sync_copy(ak[k], out.at[c, s, k])


@functools.cache
def _deg_kernel():
    return pl.kernel(
        _deg_body,
        out_type=jax.ShapeDtypeStruct((NCORE, NTAB), jnp.float32),
        mesh=plsc.VectorSubcoreMesh(core_axis_name="c", subcore_axis_name="s"),
        scratch_types=[
            pltpu.VMEM((NTAB,), jnp.float32),
            pltpu.VMEM((RPS,), jnp.float32),
            pltpu.VMEM((RPS,), jnp.float32),
            pltpu.VMEM_SHARED((NSUB, NTAB), jnp.float32),
            pltpu.VMEM((CE,), jnp.int32),
            pltpu.SemaphoreType.DMA,
        ],
        compiler_params=_sc_params(),
    )


@functools.cache
def _agg_kernel():
    return pl.kernel(
        _agg_body,
        out_type=jax.ShapeDtypeStruct((NCORE, NSUB, KPT, NTAB), jnp.float32),
        mesh=plsc.VectorSubcoreMesh(core_axis_name="c", subcore_axis_name="s"),
        scratch_types=(
            [pltpu.VMEM((NTAB,), jnp.float32) for _ in range(2 * KPT)]
            + [pltpu.VMEM((CEA,), jnp.int32), pltpu.VMEM((CEA,), jnp.int32),
               pltpu.SemaphoreType.DMA]
        ),
        compiler_params=_sc_params(),
    )


def _mm1_body(x_ref, w_ref, deg_ref, g_ref):
    m = jnp.dot(x_ref[...], w_ref[...], preferred_element_type=jnp.float32)
    dinv = lax.rsqrt(deg_ref[...] + 1.0)
    g_ref[:, 0:H1] = m[:, 0:H1] * dinv[:, 0:1]
    g_ref[:, H1:2 * H1] = m[:, H1:2 * H1] * dinv[:, 1:2]
    g_ref[:, 2 * H1:3 * H1] = m[:, 2 * H1:3 * H1]


def _comb1_body(accin, accout, g1, deg, b1i_r, b1o_r, rb1_r, h2_ref, y_ref):
    dinv = lax.rsqrt(deg[...] + 1.0)
    di = dinv[:, 0:1]
    do = dinv[:, 1:2]
    x_in = di * (accin[...] + g1[:, 0:H1]) + b1i_r[...]
    x_out = do * (accout[...] + g1[:, H1:2 * H1]) + b1o_r[...]
    h = ALPHA * x_out + (1.0 - ALPHA) * x_in + g1[:, 2 * H1:3 * H1] + rb1_r[...]
    h = jnp.maximum(h, 0.0)
    h2_ref[...] = h
    y_ref[:, 0:H1] = h * di
    y_ref[:, H1:2 * H1] = h * do


def _fin_body(accin, accout, h2, deg, w2i, w2o, r2, b2i_r, b2o_r, rb2_r,
              wf, bf_r, out_ref):
    dinv = lax.rsqrt(deg[...] + 1.0)
    di = dinv[:, 0:1]
    do = dinv[:, 1:2]
    h2v = h2[...]
    u_in = di * (accin[...] + di * h2v)
    u_out = do * (accout[...] + do * h2v)
    x_in = jnp.dot(u_in, w2i[...], preferred_element_type=jnp.float32) + b2i_r[...]
    x_out = jnp.dot(u_out, w2o[...], preferred_element_type=jnp.float32) + b2o_r[...]
    h3 = (ALPHA * x_out + (1.0 - ALPHA) * x_in
          + jnp.dot(h2v, r2[...], preferred_element_type=jnp.float32) + rb2_r[...])
    h3 = jnp.maximum(h3, 0.0)
    out_ref[...] = jnp.dot(h3, wf[...], preferred_element_type=jnp.float32) + bf_r[...]


def _row_spec(w):
    return pl.BlockSpec((RB, w), lambda i: (i, 0))


def _full_spec(shape):
    return pl.BlockSpec(shape, lambda i: tuple(0 for _ in shape))


_mm1 = pl.pallas_call(
    _mm1_body,
    grid=(GRID,),
    in_specs=[_row_spec(128), _full_spec((128, 192)), _row_spec(2)],
    out_specs=_row_spec(192),
    out_shape=jax.ShapeDtypeStruct((N, 192), jnp.float32),
)

_comb1 = pl.pallas_call(
    _comb1_body,
    grid=(GRID,),
    in_specs=[_row_spec(H1), _row_spec(H1), _row_spec(192), _row_spec(2),
              _full_spec((1, H1)), _full_spec((1, H1)), _full_spec((1, H1))],
    out_specs=[_row_spec(H1), _row_spec(2 * H1)],
    out_shape=[jax.ShapeDtypeStruct((N, H1), jnp.float32),
               jax.ShapeDtypeStruct((N, 2 * H1), jnp.float32)],
)

_fin = pl.pallas_call(
    _fin_body,
    grid=(GRID,),
    in_specs=[_row_spec(H1), _row_spec(H1), _row_spec(H1), _row_spec(2),
              _full_spec((H1, H2)), _full_spec((H1, H2)), _full_spec((H1, H2)),
              _full_spec((1, H2)), _full_spec((1, H2)), _full_spec((1, H2)),
              _full_spec((H2, NUM_CLASSES)), _full_spec((1, NUM_CLASSES))],
    out_specs=_row_spec(NUM_CLASSES),
    out_shape=jax.ShapeDtypeStruct((N, NUM_CLASSES), jnp.float32),
)


def _col_tables(gin, gout):
    gt = jnp.zeros((NCORE, H1, NTAB), jnp.float32)
    gt = gt.at[0, :, :N].set(gin.T).at[1, :, :N].set(gout.T)
    return gt.reshape(NCORE, NSUB, KPT, NTAB)


def kernel(x, edge_index, W1i, b1i, W1o, b1o, R1, rb1,
           W2i, b2i, W2o, b2o, R2, rb2, Wf, bf):
    src = edge_index[0]
    dst = edge_index[1]
    pad = jnp.full((EPAD - E,), N, jnp.int32)
    src_p = jnp.concatenate([src, pad])
    dst_p = jnp.concatenate([dst, pad])
    gidx = jnp.stack([src_p, dst_p])
    sidx = jnp.stack([dst_p, src_p])

    zvec = jnp.zeros((NTAB,), jnp.float32)

    degt = _deg_kernel()(sidx, zvec)
    degcol = jnp.stack([degt[0, :N], degt[1, :N]], axis=1)

    w1 = jnp.concatenate([W1i, W1o, R1], axis=1)
    g1 = _mm1(x, w1, degcol)

    gt1 = _col_tables(g1[:, 0:H1], g1[:, H1:2 * H1])
    acc1 = _agg_kernel()(gt1, gidx, sidx, zvec)
    acc1 = acc1.reshape(NCORE, H1, NTAB).transpose(0, 2, 1)[:, :N, :]

    h2, yc = _comb1(acc1[0], acc1[1], g1, degcol,
                    b1i.reshape(1, H1), b1o.reshape(1, H1), rb1.reshape(1, H1))

    gt2 = _col_tables(yc[:, 0:H1], yc[:, H1:2 * H1])
    acc2 = _agg_kernel()(gt2, gidx, sidx, zvec)
    acc2 = acc2.reshape(NCORE, H1, NTAB).transpose(0, 2, 1)[:, :N, :]

    return _fin(acc2[0], acc2[1], h2, degcol, W2i, W2o, R2,
                b2i.reshape(1, H2), b2o.reshape(1, H2), rb2.reshape(1, H2),
                Wf, bf.reshape(1, NUM_CLASSES))


# speedup vs baseline: 1.1633x; 1.1633x over previous
import dataclasses
import functools

import jax
import jax.numpy as jnp
from jax import lax
from jax.experimental import pallas as pl
from jax.experimental.pallas import tpu as pltpu
from jax.experimental.pallas import tpu_sc as plsc

N = 10000
E = 320000
H1 = 64
H2 = 100
NUM_CLASSES = 40
ALPHA = 0.5

NCORE = 2
NSUB = 16
NTAB = 10240
RPS = NTAB // NSUB
KPT = 4
CE = 2048
CEA = 8192
UN = 8
EPS = 20480
EPAD = EPS * NSUB

RB = 1000
GRID = N // RB


def _sc_params():
    cp = pltpu.CompilerParams()
    if "needs_layout_passes" in pltpu.CompilerParams.__dataclass_fields__:
        cp = dataclasses.replace(cp, needs_layout_passes=False)
    return cp


def _deg_body(sidx, zvec, out, hist, red, tmp, hsh, ebuf, isem):
    c = lax.axis_index("c")
    s = lax.axis_index("s")
    ones16 = jnp.ones((16,), jnp.float32)
    pltpu.sync_copy(zvec, hist)
    base = s * EPS

    def chunk(j, carry):
        pltpu.async_copy(sidx.at[c, pl.ds(base + j * CE, CE)], ebuf, isem).wait()

        @plsc.parallel_loop(0, CE // 16, unroll=8)
        def vreg(i):
            iv = ebuf[pl.ds(i * 16, 16)]
            plsc.addupdate_scatter(hist, [iv], ones16)

        return carry

    lax.fori_loop(0, EPS // CE, chunk, 0)

    pltpu.sync_copy(hist, hsh.at[s])
    plsc.subcore_barrier()

    pltpu.sync_copy(zvec.at[pl.ds(s * RPS, RPS)], red)
    for t in range(NSUB):
        pltpu.sync_copy(hsh.at[t, pl.ds(s * RPS, RPS)], tmp)

        @plsc.parallel_loop(0, RPS // 16, unroll=8)
        def addv(i):
            red[pl.ds(i * 16, 16)] = red[pl.ds(i * 16, 16)] + tmp[pl.ds(i * 16, 16)]
    pltpu.sync_copy(red, out.at[c, pl.ds(s * RPS, RPS)])


def _agg_body(gt, gidx, sidx, zvec, out, g0, g1, g2, g3, a0, a1, a2, a3,
              gbufa, sbufa, gbufb, sbufb, sema, semb):
    gk = (g0, g1, g2, g3)
    ak = (a0, a1, a2, a3)
    c = lax.axis_index("c")
    s = lax.axis_index("s")
    for k in range(KPT):
        pltpu.sync_copy(gt.at[c, s, k], gk[k])
        pltpu.sync_copy(zvec, ak[k])

    nchunk = EPAD // CEA

    def compute(gbuf, sbuf):
        @plsc.parallel_loop(0, CEA // 16, unroll=UN)
        def vreg(i):
            ivg = gbuf[pl.ds(i * 16, 16)]
            ivs = sbuf[pl.ds(i * 16, 16)]
            for k in range(KPT):
                v = plsc.load_gather(gk[k], [ivg])
                plsc.addupdate_scatter(ak[k], [ivs], v)

    def fetch(j, gbuf, sbuf, sem):
        pltpu.async_copy(gidx.at[c, pl.ds(j * CEA, CEA)], gbuf, sem)
        pltpu.async_copy(sidx.at[c, pl.ds(j * CEA, CEA)], sbuf, sem)

    def drain(gbuf, sbuf, sem):
        pltpu.make_async_copy(gidx.at[c, pl.ds(0, CEA)], gbuf, sem).wait()
        pltpu.make_async_copy(sidx.at[c, pl.ds(0, CEA)], sbuf, sem).wait()

    fetch(0, gbufa, sbufa, sema)

    def pair(h, carry):
        drain(gbufa, sbufa, sema)
        fetch(2 * h + 1, gbufb, sbufb, semb)
        compute(gbufa, sbufa)
        drain(gbufb, sbufb, semb)
        fetch(jnp.minimum(2 * h + 2, nchunk - 1), gbufa, sbufa, sema)
        compute(gbufb, sbufb)
        return carry

    lax.fori_loop(0, nchunk // 2, pair, 0)
    drain(gbufa, sbufa, sema)

    for k in range(KPT):
        pltpu.sync_copy(ak[k], out.at[c, s, k])


@functools.cache
def _deg_kernel():
    return pl.kernel(
        _deg_body,
        out_type=jax.ShapeDtypeStruct((NCORE, NTAB), jnp.float32),
        mesh=plsc.VectorSubcoreMesh(core_axis_name="c", subcore_axis_name="s"),
        scratch_types=[
            pltpu.VMEM((NTAB,), jnp.float32),
            pltpu.VMEM((RPS,), jnp.float32),
            pltpu.VMEM((RPS,), jnp.float32),
            pltpu.VMEM_SHARED((NSUB, NTAB), jnp.float32),
            pltpu.VMEM((CE,), jnp.int32),
            pltpu.SemaphoreType.DMA,
        ],
        compiler_params=_sc_params(),
    )


@functools.cache
def _agg_kernel():
    return pl.kernel(
        _agg_body,
        out_type=jax.ShapeDtypeStruct((NCORE, NSUB, KPT, NTAB), jnp.float32),
        mesh=plsc.VectorSubcoreMesh(core_axis_name="c", subcore_axis_name="s"),
        scratch_types=(
            [pltpu.VMEM((NTAB,), jnp.float32) for _ in range(2 * KPT)]
            + [pltpu.VMEM((CEA,), jnp.int32) for _ in range(4)]
            + [pltpu.SemaphoreType.DMA, pltpu.SemaphoreType.DMA]
        ),
        compiler_params=_sc_params(),
    )


def _mm1_body(x_ref, w_ref, deg_ref, g_ref):
    m = jnp.dot(x_ref[...], w_ref[...], preferred_element_type=jnp.float32)
    dinv = lax.rsqrt(deg_ref[...] + 1.0)
    g_ref[:, 0:H1] = m[:, 0:H1] * dinv[:, 0:1]
    g_ref[:, H1:2 * H1] = m[:, H1:2 * H1] * dinv[:, 1:2]
    g_ref[:, 2 * H1:3 * H1] = m[:, 2 * H1:3 * H1]


def _comb1_body(accin, accout, g1, deg, b1i_r, b1o_r, rb1_r, h2_ref, y_ref):
    dinv = lax.rsqrt(deg[...] + 1.0)
    di = dinv[:, 0:1]
    do = dinv[:, 1:2]
    x_in = di * (accin[...] + g1[:, 0:H1]) + b1i_r[...]
    x_out = do * (accout[...] + g1[:, H1:2 * H1]) + b1o_r[...]
    h = ALPHA * x_out + (1.0 - ALPHA) * x_in + g1[:, 2 * H1:3 * H1] + rb1_r[...]
    h = jnp.maximum(h, 0.0)
    h2_ref[...] = h
    y_ref[:, 0:H1] = h * di
    y_ref[:, H1:2 * H1] = h * do


def _fin_body(accin, accout, h2, deg, w2i, w2o, r2, b2i_r, b2o_r, rb2_r,
              wf, bf_r, out_ref):
    dinv = lax.rsqrt(deg[...] + 1.0)
    di = dinv[:, 0:1]
    do = dinv[:, 1:2]
    h2v = h2[...]
    u_in = di * (accin[...] + di * h2v)
    u_out = do * (accout[...] + do * h2v)
    x_in = jnp.dot(u_in, w2i[...], preferred_element_type=jnp.float32) + b2i_r[...]
    x_out = jnp.dot(u_out, w2o[...], preferred_element_type=jnp.float32) + b2o_r[...]
    h3 = (ALPHA * x_out + (1.0 - ALPHA) * x_in
          + jnp.dot(h2v, r2[...], preferred_element_type=jnp.float32) + rb2_r[...])
    h3 = jnp.maximum(h3, 0.0)
    out_ref[...] = jnp.dot(h3, wf[...], preferred_element_type=jnp.float32) + bf_r[...]


def _row_spec(w):
    return pl.BlockSpec((RB, w), lambda i: (i, 0))


def _full_spec(shape):
    return pl.BlockSpec(shape, lambda i: tuple(0 for _ in shape))


_mm1 = pl.pallas_call(
    _mm1_body,
    grid=(GRID,),
    in_specs=[_row_spec(128), _full_spec((128, 192)), _row_spec(2)],
    out_specs=_row_spec(192),
    out_shape=jax.ShapeDtypeStruct((N, 192), jnp.float32),
)

_comb1 = pl.pallas_call(
    _comb1_body,
    grid=(GRID,),
    in_specs=[_row_spec(H1), _row_spec(H1), _row_spec(192), _row_spec(2),
              _full_spec((1, H1)), _full_spec((1, H1)), _full_spec((1, H1))],
    out_specs=[_row_spec(H1), _row_spec(2 * H1)],
    out_shape=[jax.ShapeDtypeStruct((N, H1), jnp.float32),
               jax.ShapeDtypeStruct((N, 2 * H1), jnp.float32)],
)

_fin = pl.pallas_call(
    _fin_body,
    grid=(GRID,),
    in_specs=[_row_spec(H1), _row_spec(H1), _row_spec(H1), _row_spec(2),
              _full_spec((H1, H2)), _full_spec((H1, H2)), _full_spec((H1, H2)),
              _full_spec((1, H2)), _full_spec((1, H2)), _full_spec((1, H2)),
              _full_spec((H2, NUM_CLASSES)), _full_spec((1, NUM_CLASSES))],
    out_specs=_row_spec(NUM_CLASSES),
    out_shape=jax.ShapeDtypeStruct((N, NUM_CLASSES), jnp.float32),
)


def _col_tables(gin, gout):
    gt = jnp.zeros((NCORE, H1, NTAB), jnp.float32)
    gt = gt.at[0, :, :N].set(gin.T).at[1, :, :N].set(gout.T)
    return gt.reshape(NCORE, NSUB, KPT, NTAB)


def kernel(x, edge_index, W1i, b1i, W1o, b1o, R1, rb1,
           W2i, b2i, W2o, b2o, R2, rb2, Wf, bf):
    src = edge_index[0]
    dst = edge_index[1]
    pad = jnp.full((EPAD - E,), N, jnp.int32)
    src_p = jnp.concatenate([src, pad])
    dst_p = jnp.concatenate([dst, pad])
    gidx = jnp.stack([src_p, dst_p])
    sidx = jnp.stack([dst_p, src_p])

    zvec = jnp.zeros((NTAB,), jnp.float32)

    degt = _deg_kernel()(sidx, zvec)
    degcol = jnp.stack([degt[0, :N], degt[1, :N]], axis=1)

    w1 = jnp.concatenate([W1i, W1o, R1], axis=1)
    g1 = _mm1(x, w1, degcol)

    gt1 = _col_tables(g1[:, 0:H1], g1[:, H1:2 * H1])
    acc1 = _agg_kernel()(gt1, gidx, sidx, zvec)
    acc1 = acc1.reshape(NCORE, H1, NTAB).transpose(0, 2, 1)[:, :N, :]

    h2, yc = _comb1(acc1[0], acc1[1], g1, degcol,
                    b1i.reshape(1, H1), b1o.reshape(1, H1), rb1.reshape(1, H1))

    gt2 = _col_tables(yc[:, 0:H1], yc[:, H1:2 * H1])
    acc2 = _agg_kernel()(gt2, gidx, sidx, zvec)
    acc2 = acc2.reshape(NCORE, H1, NTAB).transpose(0, 2, 1)[:, :N, :]

    return _fin(acc2[0], acc2[1], h2, degcol, W2i, W2o, R2,
                b2i.reshape(1, H2), b2o.reshape(1, H2), rb2.reshape(1, H2),
                Wf, bf.reshape(1, NUM_CLASSES))


# speedup vs baseline: 1.2681x; 1.0900x over previous
import dataclasses
import functools

import jax
import jax.numpy as jnp
from jax import lax
from jax.experimental import pallas as pl
from jax.experimental.pallas import tpu as pltpu
from jax.experimental.pallas import tpu_sc as plsc

N = 10000
E = 320000
H1 = 64
H2 = 100
NUM_CLASSES = 40
ALPHA = 0.5

NCORE = 2
NSUB = 16
NTAB = 10240
RPS = NTAB // NSUB
KPT = 4
CE = 2048
CEA = 8192
UN = 8
EPS = 20480
EPAD = EPS * NSUB

RB = 1024
GRID = 10


def _sc_params():
    cp = pltpu.CompilerParams()
    if "needs_layout_passes" in pltpu.CompilerParams.__dataclass_fields__:
        cp = dataclasses.replace(cp, needs_layout_passes=False)
    return cp


def _deg_body(sidx, zvec, out, hist, red, tmp, hsh, ebuf, isem):
    c = lax.axis_index("c")
    s = lax.axis_index("s")
    ones16 = jnp.ones((16,), jnp.float32)
    pltpu.sync_copy(zvec, hist)
    base = s * EPS

    def chunk(j, carry):
        pltpu.async_copy(sidx.at[c, pl.ds(base + j * CE, CE)], ebuf, isem).wait()

        @plsc.parallel_loop(0, CE // 16, unroll=8)
        def vreg(i):
            iv = ebuf[pl.ds(i * 16, 16)]
            plsc.addupdate_scatter(hist, [iv], ones16)

        return carry

    lax.fori_loop(0, EPS // CE, chunk, 0)

    pltpu.sync_copy(hist, hsh.at[s])
    plsc.subcore_barrier()

    pltpu.sync_copy(zvec.at[pl.ds(s * RPS, RPS)], red)
    for t in range(NSUB):
        pltpu.sync_copy(hsh.at[t, pl.ds(s * RPS, RPS)], tmp)

        @plsc.parallel_loop(0, RPS // 16, unroll=8)
        def addv(i):
            red[pl.ds(i * 16, 16)] = red[pl.ds(i * 16, 16)] + tmp[pl.ds(i * 16, 16)]
    pltpu.sync_copy(red, out.at[c, pl.ds(s * RPS, RPS)])


def _agg_body(gt, gidx, sidx, zvec, out, g0, g1, g2, g3, a0, a1, a2, a3,
              gbufa, sbufa, gbufb, sbufb, sema, semb):
    gk = (g0, g1, g2, g3)
    ak = (a0, a1, a2, a3)
    c = lax.axis_index("c")
    s = lax.axis_index("s")
    for k in range(KPT):
        pltpu.sync_copy(gt.at[c, s, k], gk[k])
        pltpu.sync_copy(zvec, ak[k])

    nchunk = EPAD // CEA

    def compute(gbuf, sbuf):
        @plsc.parallel_loop(0, CEA // 16, unroll=UN)
        def vreg(i):
            ivg = gbuf[pl.ds(i * 16, 16)]
            ivs = sbuf[pl.ds(i * 16, 16)]
            for k in range(KPT):
                v = plsc.load_gather(gk[k], [ivg])
                plsc.addupdate_scatter(ak[k], [ivs], v)

    def fetch(j, gbuf, sbuf, sem):
        pltpu.async_copy(gidx.at[c, pl.ds(j * CEA, CEA)], gbuf, sem)
        pltpu.async_copy(sidx.at[c, pl.ds(j * CEA, CEA)], sbuf, sem)

    def drain(gbuf, sbuf, sem):
        pltpu.make_async_copy(gidx.at[c, pl.ds(0, CEA)], gbuf, sem).wait()
        pltpu.make_async_copy(sidx.at[c, pl.ds(0, CEA)], sbuf, sem).wait()

    fetch(0, gbufa, sbufa, sema)

    def pair(h, carry):
        drain(gbufa, sbufa, sema)
        fetch(2 * h + 1, gbufb, sbufb, semb)
        compute(gbufa, sbufa)
        drain(gbufb, sbufb, semb)
        fetch(jnp.minimum(2 * h + 2, nchunk - 1), gbufa, sbufa, sema)
        compute(gbufb, sbufb)
        return carry

    lax.fori_loop(0, nchunk // 2, pair, 0)
    drain(gbufa, sbufa, sema)

    for k in range(KPT):
        pltpu.sync_copy(ak[k], out.at[c, s, k])


@functools.cache
def _deg_kernel():
    return pl.kernel(
        _deg_body,
        out_type=jax.ShapeDtypeStruct((NCORE, NTAB), jnp.float32),
        mesh=plsc.VectorSubcoreMesh(core_axis_name="c", subcore_axis_name="s"),
        scratch_types=[
            pltpu.VMEM((NTAB,), jnp.float32),
            pltpu.VMEM((RPS,), jnp.float32),
            pltpu.VMEM((RPS,), jnp.float32),
            pltpu.VMEM_SHARED((NSUB, NTAB), jnp.float32),
            pltpu.VMEM((CE,), jnp.int32),
            pltpu.SemaphoreType.DMA,
        ],
        compiler_params=_sc_params(),
    )


@functools.cache
def _agg_kernel():
    return pl.kernel(
        _agg_body,
        out_type=jax.ShapeDtypeStruct((NCORE, NSUB, KPT, NTAB), jnp.float32),
        mesh=plsc.VectorSubcoreMesh(core_axis_name="c", subcore_axis_name="s"),
        scratch_types=(
            [pltpu.VMEM((NTAB,), jnp.float32) for _ in range(2 * KPT)]
            + [pltpu.VMEM((CEA,), jnp.int32) for _ in range(4)]
            + [pltpu.SemaphoreType.DMA, pltpu.SemaphoreType.DMA]
        ),
        compiler_params=_sc_params(),
    )


def _mm1_body(x_ref, w_ref, deg_ref, g_ref, gc_ref):
    m = jnp.dot(x_ref[...], w_ref[...], preferred_element_type=jnp.float32)
    dinv = lax.rsqrt(deg_ref[...] + 1.0)
    gin = m[:, 0:H1] * dinv[:, 0:1]
    gout = m[:, H1:2 * H1] * dinv[:, 1:2]
    g_ref[:, 0:H1] = gin
    g_ref[:, H1:2 * H1] = gout
    g_ref[:, 2 * H1:3 * H1] = m[:, 2 * H1:3 * H1]
    gc_ref[...] = jnp.transpose(jnp.concatenate([gin, gout], axis=1))


def _comb1_body(accin, accout, g1, deg, b1i_r, b1o_r, rb1_r, h2_ref, y_ref):
    dinv = lax.rsqrt(deg[...] + 1.0)
    di = dinv[:, 0:1]
    do = dinv[:, 1:2]
    x_in = di * (jnp.transpose(accin[0]) + g1[:, 0:H1]) + b1i_r[...]
    x_out = do * (jnp.transpose(accout[0]) + g1[:, H1:2 * H1]) + b1o_r[...]
    h = ALPHA * x_out + (1.0 - ALPHA) * x_in + g1[:, 2 * H1:3 * H1] + rb1_r[...]
    h = jnp.maximum(h, 0.0)
    h2_ref[...] = h
    y_ref[...] = jnp.transpose(jnp.concatenate([h * di, h * do], axis=1))


def _fin_body(accin, accout, h2, deg, w2i, w2o, r2, b2i_r, b2o_r, rb2_r,
              wf, bf_r, out_ref):
    dinv = lax.rsqrt(deg[...] + 1.0)
    di = dinv[:, 0:1]
    do = dinv[:, 1:2]
    h2v = h2[...]
    u_in = di * (jnp.transpose(accin[0]) + di * h2v)
    u_out = do * (jnp.transpose(accout[0]) + do * h2v)
    x_in = jnp.dot(u_in, w2i[...], preferred_element_type=jnp.float32) + b2i_r[...]
    x_out = jnp.dot(u_out, w2o[...], preferred_element_type=jnp.float32) + b2o_r[...]
    h3 = (ALPHA * x_out + (1.0 - ALPHA) * x_in
          + jnp.dot(h2v, r2[...], preferred_element_type=jnp.float32) + rb2_r[...])
    h3 = jnp.maximum(h3, 0.0)
    out_ref[...] = jnp.dot(h3, wf[...], preferred_element_type=jnp.float32) + bf_r[...]


def _row_spec(w):
    return pl.BlockSpec((RB, w), lambda i: (i, 0))


def _full_spec(shape):
    return pl.BlockSpec(shape, lambda i: tuple(0 for _ in shape))


def _acc_spec(core):
    return pl.BlockSpec((1, H1, RB), lambda i, c=core: (c, 0, i))


_mm1 = pl.pallas_call(
    _mm1_body,
    grid=(GRID,),
    in_specs=[_row_spec(128), _full_spec((128, 192)), _row_spec(2)],
    out_specs=[_row_spec(192), pl.BlockSpec((2 * H1, RB), lambda i: (0, i))],
    out_shape=[jax.ShapeDtypeStruct((N, 192), jnp.float32),
               jax.ShapeDtypeStruct((2 * H1, NTAB), jnp.float32)],
)

_comb1 = pl.pallas_call(
    _comb1_body,
    grid=(GRID,),
    in_specs=[_acc_spec(0), _acc_spec(1), _row_spec(192), _row_spec(2),
              _full_spec((1, H1)), _full_spec((1, H1)), _full_spec((1, H1))],
    out_specs=[_row_spec(H1), pl.BlockSpec((2 * H1, RB), lambda i: (0, i))],
    out_shape=[jax.ShapeDtypeStruct((N, H1), jnp.float32),
               jax.ShapeDtypeStruct((2 * H1, NTAB), jnp.float32)],
)

_fin = pl.pallas_call(
    _fin_body,
    grid=(GRID,),
    in_specs=[_acc_spec(0), _acc_spec(1), _row_spec(H1), _row_spec(2),
              _full_spec((H1, H2)), _full_spec((H1, H2)), _full_spec((H1, H2)),
              _full_spec((1, H2)), _full_spec((1, H2)), _full_spec((1, H2)),
              _full_spec((H2, NUM_CLASSES)), _full_spec((1, NUM_CLASSES))],
    out_specs=_row_spec(NUM_CLASSES),
    out_shape=jax.ShapeDtypeStruct((N, NUM_CLASSES), jnp.float32),
)


def kernel(x, edge_index, W1i, b1i, W1o, b1o, R1, rb1,
           W2i, b2i, W2o, b2o, R2, rb2, Wf, bf):
    src = edge_index[0]
    dst = edge_index[1]
    pad = jnp.full((EPAD - E,), N, jnp.int32)
    src_p = jnp.concatenate([src, pad])
    dst_p = jnp.concatenate([dst, pad])
    gidx = jnp.stack([src_p, dst_p])
    sidx = jnp.stack([dst_p, src_p])

    zvec = jnp.zeros((NTAB,), jnp.float32)

    degt = _deg_kernel()(sidx, zvec)
    degcol = jnp.stack([degt[0, :N], degt[1, :N]], axis=1)

    w1 = jnp.concatenate([W1i, W1o, R1], axis=1)
    g1, gcols = _mm1(x, w1, degcol)

    gt1 = gcols.reshape(NCORE, NSUB, KPT, NTAB)
    acc1 = _agg_kernel()(gt1, gidx, sidx, zvec)
    acc1 = acc1.reshape(NCORE, H1, NTAB)

    h2, ycols = _comb1(acc1, acc1, g1, degcol,
                       b1i.reshape(1, H1), b1o.reshape(1, H1), rb1.reshape(1, H1))

    gt2 = ycols.reshape(NCORE, NSUB, KPT, NTAB)
    acc2 = _agg_kernel()(gt2, gidx, sidx, zvec)
    acc2 = acc2.reshape(NCORE, H1, NTAB)

    return _fin(acc2, acc2, h2, degcol, W2i, W2o, R2,
                b2i.reshape(1, H2), b2o.reshape(1, H2), rb2.reshape(1, H2),
                Wf, bf.reshape(1, NUM_CLASSES))
